# baseline (speedup 1.0000x reference)
import functools

import jax
import jax.numpy as jnp
import numpy as np
from jax import lax
from jax.experimental import pallas as pl
from jax.experimental.pallas import tpu as pltpu
from jax.experimental.pallas import tpu_sc as plsc

B = 4096
L = 50
D = 300
DP = 384
NC = 2
NS = 16
NW = NC * NS
BAGS_W = B // NW
CB = 2
ROWS = CB * L
NCH = BAGS_W // CB

GC = 16
GB = GC * CB
NG = NCH // GC
_SIDX = (
    (np.arange(NW, dtype=np.int32)[:, None, None] // NC) * GB
    + (np.arange(NCH, dtype=np.int32)[None, :, None] % GC) * CB
    + (np.arange(ROWS, dtype=np.int32)[None, None, :] // L)
)

_ZBLK = np.zeros((GB, 128), dtype=np.float32)


@functools.cache
def _build_embbag():
    mesh = plsc.VectorSubcoreMesh(core_axis_name="c", subcore_axis_name="s")

    @functools.partial(
        pl.kernel,
        mesh=mesh,
        out_type=jax.ShapeDtypeStruct((B, DP), jnp.float32),
        scratch_types=[
            pltpu.VMEM((NCH, ROWS), jnp.int32),
            pltpu.VMEM((NCH, ROWS), jnp.int32),
            [[pltpu.VMEM((ROWS, 128), jnp.float32)
              for _ in range(3)] for _ in range(2)],
            pltpu.VMEM((GB, 128), jnp.float32),
            [pltpu.VMEM_SHARED((NS * GB, 128), jnp.float32)
             for _ in range(3)],
            pltpu.SemaphoreType.DMA,
            pltpu.SemaphoreType.DMA,
        ],
        compiler_params=pltpu.CompilerParams(use_tc_tiling_on_sc=True),
    )
    def _embbag(gidx_hbm, sidx_hbm, zero_hbm, table_hbm, tail_hbm, out_hbm,
                gidx_v, sidx_v, bufs, zbuf, accs, sem0, sem1):
        cid = lax.axis_index("c")
        sid = lax.axis_index("s")
        wid = sid * NC + cid
        base = wid * BAGS_W
        rbase = sid * GB
        sems = (sem0, sem1)
        srcs = (table_hbm.at[:, pl.ds(0, 128)],
                table_hbm.at[:, pl.ds(128, 128)],
                tail_hbm)

        pltpu.sync_copy(gidx_hbm.at[wid], gidx_v)
        pltpu.sync_copy(sidx_hbm.at[wid], sidx_v)
        pltpu.sync_copy(zero_hbm, zbuf)
        for j in range(3):
            pltpu.sync_copy(zbuf, accs[j].at[pl.ds(rbase, GB)])

        def fire(cc, slot):
            for j in range(3):
                pltpu.async_copy(srcs[j].at[gidx_v.at[cc]], bufs[slot][j],
                                 sems[slot])

        fire(0, 0)
        fire(1, 1)

        def chunk_step(cc, slot):
            for j in range(3):
                pltpu.make_async_copy(srcs[j].at[gidx_v.at[cc]],
                                      bufs[slot][j], sems[slot]).wait()
            for j in range(3):
                pltpu.sync_copy(bufs[slot][j], accs[j].at[sidx_v.at[cc]],
                                add=True)

            @pl.when(cc + 2 < NCH)
            def _():
                fire(cc + 2, slot)

        def group(g, carry):
            def pair(i, carry2):
                cc = g * GC + 2 * i
                chunk_step(cc, 0)
                chunk_step(cc + 1, 1)
                return carry2

            lax.fori_loop(0, GC // 2, pair, 0)
            for j in range(3):
                pltpu.sync_copy(accs[j].at[pl.ds(rbase, GB)],
                                out_hbm.at[pl.ds(base + g * GB, GB),
                                           pl.ds(j * 128, 128)])
                pltpu.sync_copy(zbuf, accs[j].at[pl.ds(rbase, GB)])
            return carry

        lax.fori_loop(0, NG, group, 0)

    return _embbag


def _mlp_body(x_ref, w1_ref, b1_ref, w2_ref, b2_ref, w3_ref, b3_ref,
              w4_ref, b4_ref, o_ref):
    x = x_ref[...] * np.float32(1.0 / L)
    h = jnp.dot(x, w1_ref[...], preferred_element_type=jnp.float32)
    h = jnp.maximum(h + b1_ref[...], 0.0)
    h = jnp.dot(h, w2_ref[...], preferred_element_type=jnp.float32)
    h = jnp.maximum(h + b2_ref[...], 0.0)
    h = jnp.dot(h, w3_ref[...], preferred_element_type=jnp.float32)
    h = jnp.maximum(h + b3_ref[...], 0.0)
    o = jnp.dot(h, w4_ref[...], preferred_element_type=jnp.float32)
    o_ref[...] = jax.nn.sigmoid(o + b4_ref[...])


_BT = 1024


def _mlp(x, W1, b1, W2, b2, W3, b3, W4, b4):
    full = lambda s: pl.BlockSpec(s, lambda i: (0, 0))
    return pl.pallas_call(
        _mlp_body,
        grid=(B // _BT,),
        in_specs=[
            pl.BlockSpec((_BT, DP), lambda i: (i, 0)),
            full(W1.shape), full(b1.shape),
            full(W2.shape), full(b2.shape),
            full(W3.shape), full(b3.shape),
            full(W4.shape), full(b4.shape),
        ],
        out_specs=pl.BlockSpec((_BT, 1), lambda i: (i, 0)),
        out_shape=jax.ShapeDtypeStruct((B, 1), jnp.float32),
    )(x, W1, b1, W2, b2, W3, b3, W4, b4)


def kernel(text, table, W1, b1, W2, b2, W3, b3, W4, b4):
    gidx = text.reshape(NW, NCH, ROWS)
    tail = jnp.pad(lax.slice(table, (0, 256), (100000, D)),
                   ((0, 0), (0, DP - D)))
    sums = _build_embbag()(gidx, jnp.asarray(_SIDX), jnp.asarray(_ZBLK),
                           table, tail)
    W1p = jnp.pad(W1, ((0, DP - D), (0, 0)))
    return _mlp(sums, W1p, b1.reshape(1, -1), W2, b2.reshape(1, -1),
                W3, b3.reshape(1, -1), W4, b4.reshape(1, -1))


# speedup vs baseline: 1.0811x; 1.0811x over previous
import functools

import jax
import jax.numpy as jnp
import numpy as np
from jax import lax
from jax.experimental import pallas as pl
from jax.experimental.pallas import tpu as pltpu
from jax.experimental.pallas import tpu_sc as plsc

B = 4096
L = 50
D = 300
NC = 2
NS = 16
NW = NC * NS
BAGS_W = B // NW
CB = 2
ROWS = CB * L
NCH = BAGS_W // CB

BLOCKS = ((0, 128), (128, 128), (252, 48))

GC = 16
GB = GC * CB
NG = NCH // GC
_SIDX = (
    (np.arange(NW, dtype=np.int32)[:, None, None] // NC) * GB
    + (np.arange(NCH, dtype=np.int32)[None, :, None] % GC) * CB
    + (np.arange(ROWS, dtype=np.int32)[None, None, :] // L)
)


@functools.cache
def _build_embbag(w):
    mesh = plsc.VectorSubcoreMesh(core_axis_name="c", subcore_axis_name="s")

    @functools.partial(
        pl.kernel,
        mesh=mesh,
        out_type=jax.ShapeDtypeStruct((B, w), jnp.float32),
        scratch_types=[
            pltpu.VMEM((NCH, ROWS), jnp.int32),
            pltpu.VMEM((NCH, ROWS), jnp.int32),
            [pltpu.VMEM((ROWS, w), jnp.float32)
             for _ in range(2)],
            pltpu.VMEM((GB, w), jnp.float32),
            pltpu.VMEM_SHARED((NS * GB, w), jnp.float32),
            pltpu.SemaphoreType.DMA,
            pltpu.SemaphoreType.DMA,
        ],
        compiler_params=pltpu.CompilerParams(use_tc_tiling_on_sc=(w % 128 == 0)),
    )
    def _embbag(gidx_hbm, sidx_hbm, zero_hbm, tab_hbm, out_hbm,
                gidx_v, sidx_v, bufs, zbuf, acc, sem0, sem1):
        cid = lax.axis_index("c")
        sid = lax.axis_index("s")
        wid = sid * NC + cid
        base = wid * BAGS_W
        rbase = sid * GB
        sems = (sem0, sem1)

        pltpu.sync_copy(gidx_hbm.at[wid], gidx_v)
        pltpu.sync_copy(sidx_hbm.at[wid], sidx_v)
        pltpu.sync_copy(zero_hbm, zbuf)
        pltpu.sync_copy(zbuf, acc.at[pl.ds(rbase, GB)])

        def fire(cc, slot):
            pltpu.async_copy(tab_hbm.at[gidx_v.at[cc]], bufs[slot],
                             sems[slot])

        fire(0, 0)
        fire(1, 1)

        def chunk_step(cc, slot):
            pltpu.make_async_copy(tab_hbm.at[gidx_v.at[cc]],
                                  bufs[slot], sems[slot]).wait()
            pltpu.sync_copy(bufs[slot], acc.at[sidx_v.at[cc]], add=True)

            @pl.when(cc + 2 < NCH)
            def _():
                fire(cc + 2, slot)

        def group(g, carry):
            def pair(i, carry2):
                cc = g * GC + 2 * i
                chunk_step(cc, 0)
                chunk_step(cc + 1, 1)
                return carry2

            lax.fori_loop(0, GC // 2, pair, 0)
            pltpu.sync_copy(acc.at[pl.ds(rbase, GB)],
                            out_hbm.at[pl.ds(base + g * GB, GB)])
            pltpu.sync_copy(zbuf, acc.at[pl.ds(rbase, GB)])
            return carry

        lax.fori_loop(0, NG, group, 0)

    return _embbag


def _mlp_body(x0_ref, x1_ref, x2_ref, w1a_ref, w1b_ref, w1c_ref, b1_ref,
              w2_ref, b2_ref, w3_ref, b3_ref, w4_ref, b4_ref, o_ref):
    s = np.float32(1.0 / L)
    h = jnp.dot(x0_ref[...] * s, w1a_ref[...],
                preferred_element_type=jnp.float32)
    h += jnp.dot(x1_ref[...] * s, w1b_ref[...],
                 preferred_element_type=jnp.float32)
    h += jnp.dot(x2_ref[...] * s, w1c_ref[...],
                 preferred_element_type=jnp.float32)
    h = jnp.maximum(h + b1_ref[...], 0.0)
    h = jnp.dot(h, w2_ref[...], preferred_element_type=jnp.float32)
    h = jnp.maximum(h + b2_ref[...], 0.0)
    h = jnp.dot(h, w3_ref[...], preferred_element_type=jnp.float32)
    h = jnp.maximum(h + b3_ref[...], 0.0)
    o = jnp.dot(h, w4_ref[...], preferred_element_type=jnp.float32)
    o_ref[...] = jax.nn.sigmoid(o + b4_ref[...])


_BT = 1024


def _mlp(x0, x1, x2, W1a, W1b, W1c, b1, W2, b2, W3, b3, W4, b4):
    full = lambda a: pl.BlockSpec(a.shape, lambda i: (0, 0))
    xspec = lambda a: pl.BlockSpec((_BT, a.shape[1]), lambda i: (i, 0))
    return pl.pallas_call(
        _mlp_body,
        grid=(B // _BT,),
        in_specs=[
            xspec(x0), xspec(x1), xspec(x2),
            full(W1a), full(W1b), full(W1c), full(b1),
            full(W2), full(b2), full(W3), full(b3), full(W4), full(b4),
        ],
        out_specs=pl.BlockSpec((_BT, 1), lambda i: (i, 0)),
        out_shape=jax.ShapeDtypeStruct((B, 1), jnp.float32),
    )(x0, x1, x2, W1a, W1b, W1c, b1, W2, b2, W3, b3, W4, b4)


def kernel(text, table, W1, b1, W2, b2, W3, b3, W4, b4):
    gidx = text.reshape(NW, NCH, ROWS)
    sidx = jnp.asarray(_SIDX)
    sums = []
    for c0, w in BLOCKS:
        blk = lax.slice(table, (0, c0), (100000, c0 + w))
        sums.append(
            _build_embbag(w)(gidx, sidx, jnp.zeros((GB, w), jnp.float32),
                             blk))
    W1a = lax.slice(W1, (0, 0), (128, 1000))
    W1b = lax.slice(W1, (128, 0), (256, 1000))
    W1c = jnp.pad(lax.slice(W1, (256, 0), (300, 1000)), ((4, 0), (0, 0)))
    return _mlp(sums[0], sums[1], sums[2], W1a, W1b, W1c,
                b1.reshape(1, -1), W2, b2.reshape(1, -1),
                W3, b3.reshape(1, -1), W4, b4.reshape(1, -1))


# speedup vs baseline: 1.1186x; 1.0347x over previous
import functools

import jax
import jax.numpy as jnp
import numpy as np
from jax import lax
from jax.experimental import pallas as pl
from jax.experimental.pallas import tpu as pltpu
from jax.experimental.pallas import tpu_sc as plsc

B = 4096
L = 50
D = 300
NC = 2
NS = 16
NW = NC * NS
BAGS_W = B // NW
CB = 2
ROWS = CB * L
NCH = BAGS_W // CB

BLOCKS = ((0, 128), (128, 128), (252, 48))

GC = 16
GB = GC * CB
NG = NCH // GC
_SIDX = (
    (np.arange(NW, dtype=np.int32)[:, None, None] // NC) * GB
    + (np.arange(NCH, dtype=np.int32)[None, :, None] % GC) * CB
    + (np.arange(ROWS, dtype=np.int32)[None, None, :] // L)
)


@functools.cache
def _build_embbag(w):
    mesh = plsc.VectorSubcoreMesh(core_axis_name="c", subcore_axis_name="s")

    @functools.partial(
        pl.kernel,
        mesh=mesh,
        out_type=jax.ShapeDtypeStruct((B, w), jnp.float32),
        scratch_types=[
            pltpu.VMEM((NCH, ROWS), jnp.int32),
            pltpu.VMEM((NCH, ROWS), jnp.int32),
            [pltpu.VMEM((ROWS, w), jnp.float32)
             for _ in range(4)],
            pltpu.VMEM((GB, w), jnp.float32),
            pltpu.VMEM_SHARED((NS * GB, w), jnp.float32),
            [pltpu.SemaphoreType.DMA for _ in range(4)],
        ],
        compiler_params=pltpu.CompilerParams(use_tc_tiling_on_sc=(w % 128 == 0)),
    )
    def _embbag(gidx_hbm, sidx_hbm, zero_hbm, tab_hbm, out_hbm,
                gidx_v, sidx_v, bufs, zbuf, acc, sems):
        cid = lax.axis_index("c")
        sid = lax.axis_index("s")
        wid = sid * NC + cid
        base = wid * BAGS_W
        rbase = sid * GB

        pltpu.sync_copy(gidx_hbm.at[wid], gidx_v)
        pltpu.sync_copy(sidx_hbm.at[wid], sidx_v)
        pltpu.sync_copy(zero_hbm, zbuf)
        pltpu.sync_copy(zbuf, acc.at[pl.ds(rbase, GB)])

        def fire(cc, slot):
            pltpu.async_copy(tab_hbm.at[gidx_v.at[cc]], bufs[slot],
                             sems[slot])

        NSLOT = len(bufs)

        for s in range(NSLOT):
            fire(s, s)

        def chunk_step(cc, slot):
            pltpu.make_async_copy(tab_hbm.at[gidx_v.at[cc]],
                                  bufs[slot], sems[slot]).wait()
            pltpu.sync_copy(bufs[slot], acc.at[sidx_v.at[cc]], add=True)

            @pl.when(cc + NSLOT < NCH)
            def _():
                fire(cc + NSLOT, slot)

        def group(g, carry):
            def quad(i, carry2):
                cc = g * GC + NSLOT * i
                for s in range(NSLOT):
                    chunk_step(cc + s, s)
                return carry2

            lax.fori_loop(0, GC // NSLOT, quad, 0)
            pltpu.sync_copy(acc.at[pl.ds(rbase, GB)],
                            out_hbm.at[pl.ds(base + g * GB, GB)])
            pltpu.sync_copy(zbuf, acc.at[pl.ds(rbase, GB)])
            return carry

        lax.fori_loop(0, NG, group, 0)

    return _embbag


def _mlp_body(x0_ref, x1_ref, x2_ref, w1a_ref, w1b_ref, w1c_ref, b1_ref,
              w2_ref, b2_ref, w3_ref, b3_ref, w4_ref, b4_ref, o_ref):
    s = np.float32(1.0 / L)
    h = jnp.dot(x0_ref[...] * s, w1a_ref[...],
                preferred_element_type=jnp.float32)
    h += jnp.dot(x1_ref[...] * s, w1b_ref[...],
                 preferred_element_type=jnp.float32)
    h += jnp.dot(x2_ref[...] * s, w1c_ref[...],
                 preferred_element_type=jnp.float32)
    h = jnp.maximum(h + b1_ref[...], 0.0)
    h = jnp.dot(h, w2_ref[...], preferred_element_type=jnp.float32)
    h = jnp.maximum(h + b2_ref[...], 0.0)
    h = jnp.dot(h, w3_ref[...], preferred_element_type=jnp.float32)
    h = jnp.maximum(h + b3_ref[...], 0.0)
    o = jnp.dot(h, w4_ref[...], preferred_element_type=jnp.float32)
    o_ref[...] = jax.nn.sigmoid(o + b4_ref[...])


_BT = 1024


def _mlp(x0, x1, x2, W1a, W1b, W1c, b1, W2, b2, W3, b3, W4, b4):
    full = lambda a: pl.BlockSpec(a.shape, lambda i: (0, 0))
    xspec = lambda a: pl.BlockSpec((_BT, a.shape[1]), lambda i: (i, 0))
    return pl.pallas_call(
        _mlp_body,
        grid=(B // _BT,),
        in_specs=[
            xspec(x0), xspec(x1), xspec(x2),
            full(W1a), full(W1b), full(W1c), full(b1),
            full(W2), full(b2), full(W3), full(b3), full(W4), full(b4),
        ],
        out_specs=pl.BlockSpec((_BT, 1), lambda i: (i, 0)),
        out_shape=jax.ShapeDtypeStruct((B, 1), jnp.float32),
    )(x0, x1, x2, W1a, W1b, W1c, b1, W2, b2, W3, b3, W4, b4)


def kernel(text, table, W1, b1, W2, b2, W3, b3, W4, b4):
    gidx = text.reshape(NW, NCH, ROWS)
    sidx = jnp.asarray(_SIDX)
    sums = []
    for c0, w in BLOCKS:
        blk = lax.slice(table, (0, c0), (100000, c0 + w))
        sums.append(
            _build_embbag(w)(gidx, sidx, jnp.zeros((GB, w), jnp.float32),
                             blk))
    W1a = lax.slice(W1, (0, 0), (128, 1000))
    W1b = lax.slice(W1, (128, 0), (256, 1000))
    W1c = jnp.pad(lax.slice(W1, (256, 0), (300, 1000)), ((4, 0), (0, 0)))
    return _mlp(sums[0], sums[1], sums[2], W1a, W1b, W1c,
                b1.reshape(1, -1), W2, b2.reshape(1, -1),
                W3, b3.reshape(1, -1), W4, b4.reshape(1, -1))


# speedup vs baseline: 1.1648x; 1.0413x over previous
import functools

import jax
import jax.numpy as jnp
import numpy as np
from jax import lax
from jax.experimental import pallas as pl
from jax.experimental.pallas import tpu as pltpu
from jax.experimental.pallas import tpu_sc as plsc

B = 4096
L = 50
D = 300
NC = 2
NS = 16
NW = NC * NS
BAGS_W = B // NW
CB = 2
ROWS = CB * L
NCH = BAGS_W // CB

BLOCKS = ((0, 128), (128, 128), (252, 48))

GC = 16
GB = GC * CB
NG = NCH // GC
_SIDX = (
    (np.arange(NW, dtype=np.int32)[:, None, None] // NC) * GB
    + (np.arange(NCH, dtype=np.int32)[None, :, None] % GC) * CB
    + (np.arange(ROWS, dtype=np.int32)[None, None, :] // L)
)


@functools.cache
def _build_embbag(w):
    mesh = plsc.VectorSubcoreMesh(core_axis_name="c", subcore_axis_name="s")

    @functools.partial(
        pl.kernel,
        mesh=mesh,
        out_type=jax.ShapeDtypeStruct((B, w), jnp.float32),
        scratch_types=[
            pltpu.VMEM((NCH, ROWS), jnp.int32),
            pltpu.VMEM((NCH, ROWS), jnp.int32),
            [pltpu.VMEM((ROWS, w), jnp.float32)
             for _ in range(8)],
            pltpu.VMEM((GB, w), jnp.float32),
            pltpu.VMEM_SHARED((NS * GB, w), jnp.float32),
            [pltpu.SemaphoreType.DMA for _ in range(8)],
            [pltpu.SemaphoreType.DMA for _ in range(8)],
        ],
        compiler_params=pltpu.CompilerParams(use_tc_tiling_on_sc=(w % 128 == 0)),
    )
    def _embbag(gidx_hbm, sidx_hbm, zero_hbm, tab_hbm, out_hbm,
                gidx_v, sidx_v, bufs, zbuf, acc, gsems, ssems):
        cid = lax.axis_index("c")
        sid = lax.axis_index("s")
        wid = sid * NC + cid
        base = wid * BAGS_W
        rbase = sid * GB
        NSLOT = len(bufs)
        AHEAD = 6

        pltpu.sync_copy(gidx_hbm.at[wid], gidx_v)
        pltpu.sync_copy(sidx_hbm.at[wid], sidx_v)
        pltpu.sync_copy(zero_hbm, zbuf)
        pltpu.sync_copy(zbuf, acc.at[pl.ds(rbase, GB)])

        def fire(m, s):
            pltpu.async_copy(tab_hbm.at[gidx_v.at[m]], bufs[s], gsems[s])

        def scat(cc, s):
            return pltpu.make_async_copy(bufs[s], acc.at[sidx_v.at[cc]],
                                         ssems[s])

        for m in range(AHEAD):
            fire(m, m % NSLOT)

        def group(g, carry):
            gbase = g * GC
            for i in range(GC):
                cc = gbase + i
                s = i % NSLOT
                pltpu.make_async_copy(tab_hbm.at[gidx_v.at[cc]],
                                      bufs[s], gsems[s]).wait()
                scat(cc, s).start()
                m = cc + AHEAD
                if i >= 2:
                    scat(cc - 2, (i - 2) % NSLOT).wait()

                @pl.when(m < NCH)
                def _():
                    fire(m, (i + AHEAD) % NSLOT)

            scat(gbase + GC - 2, (GC - 2) % NSLOT).wait()
            scat(gbase + GC - 1, (GC - 1) % NSLOT).wait()
            pltpu.sync_copy(acc.at[pl.ds(rbase, GB)],
                            out_hbm.at[pl.ds(base + g * GB, GB)])
            pltpu.sync_copy(zbuf, acc.at[pl.ds(rbase, GB)])
            return carry

        lax.fori_loop(0, NG, group, 0)

    return _embbag


def _mlp_body(x0_ref, x1_ref, x2_ref, w1a_ref, w1b_ref, w1c_ref, b1_ref,
              w2_ref, b2_ref, w3_ref, b3_ref, w4_ref, b4_ref, o_ref):
    s = np.float32(1.0 / L)
    h = jnp.dot(x0_ref[...] * s, w1a_ref[...],
                preferred_element_type=jnp.float32)
    h += jnp.dot(x1_ref[...] * s, w1b_ref[...],
                 preferred_element_type=jnp.float32)
    h += jnp.dot(x2_ref[...] * s, w1c_ref[...],
                 preferred_element_type=jnp.float32)
    h = jnp.maximum(h + b1_ref[...], 0.0)
    h = jnp.dot(h, w2_ref[...], preferred_element_type=jnp.float32)
    h = jnp.maximum(h + b2_ref[...], 0.0)
    h = jnp.dot(h, w3_ref[...], preferred_element_type=jnp.float32)
    h = jnp.maximum(h + b3_ref[...], 0.0)
    o = jnp.dot(h, w4_ref[...], preferred_element_type=jnp.float32)
    o_ref[...] = jax.nn.sigmoid(o + b4_ref[...])


_BT = 1024


def _mlp(x0, x1, x2, W1a, W1b, W1c, b1, W2, b2, W3, b3, W4, b4):
    full = lambda a: pl.BlockSpec(a.shape, lambda i: (0, 0))
    xspec = lambda a: pl.BlockSpec((_BT, a.shape[1]), lambda i: (i, 0))
    return pl.pallas_call(
        _mlp_body,
        grid=(B // _BT,),
        in_specs=[
            xspec(x0), xspec(x1), xspec(x2),
            full(W1a), full(W1b), full(W1c), full(b1),
            full(W2), full(b2), full(W3), full(b3), full(W4), full(b4),
        ],
        out_specs=pl.BlockSpec((_BT, 1), lambda i: (i, 0)),
        out_shape=jax.ShapeDtypeStruct((B, 1), jnp.float32),
    )(x0, x1, x2, W1a, W1b, W1c, b1, W2, b2, W3, b3, W4, b4)


def kernel(text, table, W1, b1, W2, b2, W3, b3, W4, b4):
    gidx = text.reshape(NW, NCH, ROWS)
    sidx = jnp.asarray(_SIDX)
    sums = []
    for c0, w in BLOCKS:
        blk = lax.slice(table, (0, c0), (100000, c0 + w))
        sums.append(
            _build_embbag(w)(gidx, sidx, jnp.zeros((GB, w), jnp.float32),
                             blk))
    W1a = lax.slice(W1, (0, 0), (128, 1000))
    W1b = lax.slice(W1, (128, 0), (256, 1000))
    W1c = jnp.pad(lax.slice(W1, (256, 0), (300, 1000)), ((4, 0), (0, 0)))
    return _mlp(sums[0], sums[1], sums[2], W1a, W1b, W1c,
                b1.reshape(1, -1), W2, b2.reshape(1, -1),
                W3, b3.reshape(1, -1), W4, b4.reshape(1, -1))
